# Initial kernel scaffold; baseline (speedup 1.0000x reference)
import jax
import jax.numpy as jnp
from jax.experimental import pallas as pl


def kernel(loc_preds, conf_preds, gt_boxes, gt_labels, default_boxes):
    raise NotImplementedError("write your pallas kernel here")


# speedup vs baseline: 14.5487x; 14.5487x over previous
import functools

import jax
import jax.numpy as jnp
from jax.experimental import pallas as pl

_NUM_CLASSES = 21
_NUM_PRIORS = 8732
_BATCH = 32
_NUM_GT = 10
_PAD = 8832
_CPAD = 24
_NEG_INF = -1e30


def _loss_kernel(loc_ref, conf_ref, gt_ref, db_ref, num_ref, npos_ref):
    b = pl.program_id(0)

    lane = jax.lax.broadcasted_iota(jnp.int32, (1, _PAD), 1)
    lane_valid = lane < _NUM_PRIORS

    db = db_ref[...]
    dbx0 = db[0:1, :]
    dby0 = db[1:2, :]
    dbx1 = db[2:3, :]
    dby1 = db[3:4, :]

    gt = gt_ref[0]
    gx0 = gt[:, 0:1]
    gy0 = gt[:, 1:2]
    gx1 = gt[:, 2:3]
    gy1 = gt[:, 3:4]

    area_d = (dbx1 - dbx0) * (dby1 - dby0)
    area_g = (gx1 - gx0) * (gy1 - gy0)
    ltx = jnp.maximum(dbx0, gx0)
    lty = jnp.maximum(dby0, gy0)
    rbx = jnp.minimum(dbx1, gx1)
    rby = jnp.minimum(dby1, gy1)
    iw = jnp.maximum(rbx - ltx, 0.0)
    ih = jnp.maximum(rby - lty, 0.0)
    inter = iw * ih
    union = area_d + area_g - inter
    iou = inter / union
    iou = jnp.where(lane_valid, iou, -1.0)

    g_col = jax.lax.broadcasted_iota(jnp.int32, (_NUM_GT, _PAD), 0)
    p_row = jax.lax.broadcasted_iota(jnp.int32, (_NUM_GT, _PAD), 1)

    best_ov = jnp.max(iou, axis=0, keepdims=True)
    bt_idx = jnp.min(jnp.where(iou == best_ov, g_col, _NUM_GT),
                     axis=0, keepdims=True)

    m_g = jnp.max(iou, axis=1, keepdims=True)
    bpi = jnp.min(jnp.where(iou == m_g, p_row, jnp.int32(1 << 30)),
                  axis=1, keepdims=True)

    ov_hit = p_row == bpi
    ocand = jnp.max(jnp.where(ov_hit, g_col, -1), axis=0, keepdims=True)
    bt_idx = jnp.where(ocand >= 0, ocand, bt_idx)
    best_ov = jnp.where(ocand >= 0, 2.0, best_ov)

    pos = best_ov >= 0.5
    npos_i = jnp.sum(pos.astype(jnp.int32))

    sel = g_col == bt_idx
    zf = jnp.zeros((), jnp.float32)
    mx0 = jnp.sum(jnp.where(sel, gx0, zf), axis=0, keepdims=True)
    my0 = jnp.sum(jnp.where(sel, gy0, zf), axis=0, keepdims=True)
    mx1 = jnp.sum(jnp.where(sel, gx1, zf), axis=0, keepdims=True)
    my1 = jnp.sum(jnp.where(sel, gy1, zf), axis=0, keepdims=True)

    g_w = mx1 - mx0
    g_h = my1 - my0
    g_cx = mx0 + g_w * 0.5
    g_cy = my0 + g_h * 0.5
    d_w = dbx1 - dbx0
    d_h = dby1 - dby0
    d_cx = dbx0 + d_w * 0.5
    d_cy = dby0 + d_h * 0.5
    dwe = d_w + 1e-8
    dhe = d_h + 1e-8
    t0 = (g_cx - d_cx) / dwe
    t1 = (g_cy - d_cy) / dhe
    t2 = jnp.log(g_w / dwe)
    t3 = jnp.log(g_h / dhe)

    lp = loc_ref[0]

    def _sl1(d):
        return jnp.where(d < 1.0, 0.5 * d * d, d - 0.5)

    sl1 = (_sl1(jnp.abs(lp[0:1, :] - t0)) + _sl1(jnp.abs(lp[1:2, :] - t1))
           + _sl1(jnp.abs(lp[2:3, :] - t2)) + _sl1(jnp.abs(lp[3:4, :] - t3)))
    loc_loss = jnp.sum(jnp.where(pos, sl1, zf))

    cf = conf_ref[0]
    cmax = jnp.max(cf, axis=0, keepdims=True)
    ssum = jnp.sum(jnp.exp(cf - cmax), axis=0, keepdims=True)
    lse = cmax + jnp.log(ssum)
    closs = lse - cf[0:1, :]
    closs = jnp.where(jnp.logical_or(pos, jnp.logical_not(lane_valid)),
                      zf, closs)

    k = jnp.minimum(3 * npos_i, _NUM_PRIORS - 1)
    vi = jax.lax.bitcast_convert_type(closs, jnp.int32)

    def _body(_, lohi):
        lo, hi = lohi
        mid = lo + (hi - lo) // 2
        cnt = jnp.sum((vi > mid).astype(jnp.int32))
        big = cnt >= k
        return (jnp.where(big, mid, lo), jnp.where(big, hi, mid))

    lo0 = jnp.int32(-1)
    hi0 = jnp.int32(0x7F800000)
    _, kth = jax.lax.fori_loop(0, 31, _body, (lo0, hi0))
    cnt_gt = jnp.sum((vi > kth).astype(jnp.int32))
    sum_gt = jnp.sum(jnp.where(vi > kth, closs, zf))
    kth_f = jax.lax.bitcast_convert_type(kth, jnp.float32)
    neg_loss = sum_gt + kth_f * (k - cnt_gt).astype(jnp.float32)
    neg_loss = jnp.where(k > 0, neg_loss, zf)

    num_img = loc_loss + neg_loss
    np_img = npos_i.astype(jnp.float32)

    num2 = jnp.reshape(num_img, (1, 1))
    np2 = jnp.reshape(np_img, (1, 1))

    @pl.when(b == 0)
    def _init():
        num_ref[:, :] = num2
        npos_ref[:, :] = np2

    @pl.when(b != 0)
    def _acc():
        num_ref[:, :] += num2
        npos_ref[:, :] += np2


@functools.partial(jax.jit, static_argnames=("interpret",))
def _ssd_loss(loc_preds, conf_preds, gt_boxes, default_boxes, interpret=False):
    locp = jnp.transpose(loc_preds, (0, 2, 1))
    locp = jnp.pad(locp, ((0, 0), (0, 0), (0, _PAD - _NUM_PRIORS)))
    confp = jnp.transpose(conf_preds, (0, 2, 1))
    confp = jnp.pad(confp,
                    ((0, 0), (0, _CPAD - _NUM_CLASSES), (0, _PAD - _NUM_PRIORS)),
                    constant_values=_NEG_INF)
    dbp = jnp.pad(jnp.transpose(default_boxes, (1, 0)),
                  ((0, 0), (0, _PAD - _NUM_PRIORS)))

    num, npos = pl.pallas_call(
        _loss_kernel,
        grid=(_BATCH,),
        in_specs=[
            pl.BlockSpec((1, 4, _PAD), lambda b: (b, 0, 0)),
            pl.BlockSpec((1, _CPAD, _PAD), lambda b: (b, 0, 0)),
            pl.BlockSpec((1, _NUM_GT, 4), lambda b: (b, 0, 0)),
            pl.BlockSpec((4, _PAD), lambda b: (0, 0)),
        ],
        out_specs=[
            pl.BlockSpec((1, 1), lambda b: (0, 0)),
            pl.BlockSpec((1, 1), lambda b: (0, 0)),
        ],
        out_shape=[
            jax.ShapeDtypeStruct((1, 1), jnp.float32),
            jax.ShapeDtypeStruct((1, 1), jnp.float32),
        ],
        interpret=interpret,
    )(locp, confp, gt_boxes, dbp)

    return num[0, 0] / (npos[0, 0] + 1e-6)


def kernel(loc_preds, conf_preds, gt_boxes, gt_labels, default_boxes):
    del gt_labels
    return _ssd_loss(loc_preds, conf_preds, gt_boxes, default_boxes)


# speedup vs baseline: 30.7046x; 2.1105x over previous
import functools

import jax
import jax.numpy as jnp
from jax.experimental import pallas as pl
from jax.experimental.pallas import tpu as pltpu

_NUM_CLASSES = 21
_NUM_PRIORS = 8732
_BATCH = 32
_NUM_GT = 10
_PAD = 8832
_CPAD = 24
_NEG_INF = -1e30


def _loss_kernel(loc_ref, conf_ref, gt_ref, db_ref, num_ref, npos_ref,
                 closs_ref, kcol_ref):
    b = pl.program_id(0)

    lane = jax.lax.broadcasted_iota(jnp.int32, (1, _PAD), 1)
    lane_valid = lane < _NUM_PRIORS

    db = db_ref[...]
    dbx0 = db[0:1, :]
    dby0 = db[1:2, :]
    dbx1 = db[2:3, :]
    dby1 = db[3:4, :]

    gt = gt_ref[0]
    gx0 = gt[:, 0:1]
    gy0 = gt[:, 1:2]
    gx1 = gt[:, 2:3]
    gy1 = gt[:, 3:4]

    area_d = (dbx1 - dbx0) * (dby1 - dby0)
    area_g = (gx1 - gx0) * (gy1 - gy0)
    ltx = jnp.maximum(dbx0, gx0)
    lty = jnp.maximum(dby0, gy0)
    rbx = jnp.minimum(dbx1, gx1)
    rby = jnp.minimum(dby1, gy1)
    iw = jnp.maximum(rbx - ltx, 0.0)
    ih = jnp.maximum(rby - lty, 0.0)
    inter = iw * ih
    union = area_d + area_g - inter
    iou = inter / union
    iou = jnp.where(lane_valid, iou, -1.0)

    g_col = jax.lax.broadcasted_iota(jnp.int32, (_NUM_GT, _PAD), 0)
    p_row = jax.lax.broadcasted_iota(jnp.int32, (_NUM_GT, _PAD), 1)

    best_ov = jnp.max(iou, axis=0, keepdims=True)
    bt_idx = jnp.min(jnp.where(iou == best_ov, g_col, _NUM_GT),
                     axis=0, keepdims=True)

    m_g = jnp.max(iou, axis=1, keepdims=True)
    bpi = jnp.min(jnp.where(iou == m_g, p_row, jnp.int32(1 << 30)),
                  axis=1, keepdims=True)

    ov_hit = p_row == bpi
    ocand = jnp.max(jnp.where(ov_hit, g_col, -1), axis=0, keepdims=True)
    bt_idx = jnp.where(ocand >= 0, ocand, bt_idx)
    best_ov = jnp.where(ocand >= 0, 2.0, best_ov)

    pos = best_ov >= 0.5
    npos_i = jnp.sum(pos.astype(jnp.int32))

    sel = g_col == bt_idx
    zf = jnp.zeros((), jnp.float32)
    mx0 = jnp.sum(jnp.where(sel, gx0, zf), axis=0, keepdims=True)
    my0 = jnp.sum(jnp.where(sel, gy0, zf), axis=0, keepdims=True)
    mx1 = jnp.sum(jnp.where(sel, gx1, zf), axis=0, keepdims=True)
    my1 = jnp.sum(jnp.where(sel, gy1, zf), axis=0, keepdims=True)

    g_w = mx1 - mx0
    g_h = my1 - my0
    g_cx = mx0 + g_w * 0.5
    g_cy = my0 + g_h * 0.5
    d_w = dbx1 - dbx0
    d_h = dby1 - dby0
    d_cx = dbx0 + d_w * 0.5
    d_cy = dby0 + d_h * 0.5
    dwe = d_w + 1e-8
    dhe = d_h + 1e-8
    t0 = (g_cx - d_cx) / dwe
    t1 = (g_cy - d_cy) / dhe
    t2 = jnp.log(g_w / dwe)
    t3 = jnp.log(g_h / dhe)

    lp = loc_ref[0]

    def _sl1(d):
        return jnp.where(d < 1.0, 0.5 * d * d, d - 0.5)

    sl1 = (_sl1(jnp.abs(lp[0:1, :] - t0)) + _sl1(jnp.abs(lp[1:2, :] - t1))
           + _sl1(jnp.abs(lp[2:3, :] - t2)) + _sl1(jnp.abs(lp[3:4, :] - t3)))
    loc_loss = jnp.sum(jnp.where(pos, sl1, zf))

    cf = conf_ref[0]
    cmax = jnp.max(cf, axis=0, keepdims=True)
    ssum = jnp.sum(jnp.exp(cf - cmax), axis=0, keepdims=True)
    lse = cmax + jnp.log(ssum)
    closs = lse - cf[0:1, :]
    closs = jnp.where(jnp.logical_or(pos, jnp.logical_not(lane_valid)),
                      zf, closs)

    np_img = npos_i.astype(jnp.float32)
    closs_ref[pl.ds(b, 1), :] = closs
    kcol_ref[pl.ds(b, 1), :] = jnp.broadcast_to(jnp.reshape(np_img, (1, 1)),
                                                (1, 128))

    num2 = jnp.reshape(loc_loss, (1, 1))
    np2 = jnp.reshape(np_img, (1, 1))

    @pl.when(b == 0)
    def _init():
        num_ref[:, :] = num2
        npos_ref[:, :] = np2

    @pl.when(b != 0)
    def _acc():
        num_ref[:, :] += num2
        npos_ref[:, :] += np2

    @pl.when(b == _BATCH - 1)
    def _neg_mine():
        v = closs_ref[...]
        vi = jax.lax.bitcast_convert_type(v, jnp.int32)
        npv = kcol_ref[:, 0:1].astype(jnp.int32)
        k = jnp.minimum(3 * npv, _NUM_PRIORS - 1)

        def _body(_, lohi):
            lo, hi = lohi
            mid = lo + (hi - lo) // 2
            cnt = jnp.sum((vi > mid).astype(jnp.int32), axis=1, keepdims=True)
            big = cnt >= k
            return (jnp.where(big, mid, lo), jnp.where(big, hi, mid))

        lo0 = jnp.full((_BATCH, 1), -1, jnp.int32)
        hi0 = jnp.full((_BATCH, 1), 0x7F800000, jnp.int32)
        _, kth = jax.lax.fori_loop(0, 31, _body, (lo0, hi0))
        gt_mask = vi > kth
        cnt_gt = jnp.sum(gt_mask.astype(jnp.int32), axis=1, keepdims=True)
        sum_gt = jnp.sum(jnp.where(gt_mask, v, 0.0), axis=1, keepdims=True)
        kth_f = jax.lax.bitcast_convert_type(kth, jnp.float32)
        neg = sum_gt + kth_f * (k - cnt_gt).astype(jnp.float32)
        neg = jnp.where(k > 0, neg, 0.0)
        num_ref[:, :] += jnp.reshape(jnp.sum(neg), (1, 1))


@functools.partial(jax.jit, static_argnames=("interpret",))
def _ssd_loss(loc_preds, conf_preds, gt_boxes, default_boxes, interpret=False):
    locp = jnp.transpose(loc_preds, (0, 2, 1))
    locp = jnp.pad(locp, ((0, 0), (0, 0), (0, _PAD - _NUM_PRIORS)))
    confp = jnp.transpose(conf_preds, (0, 2, 1))
    confp = jnp.pad(confp,
                    ((0, 0), (0, _CPAD - _NUM_CLASSES), (0, _PAD - _NUM_PRIORS)),
                    constant_values=_NEG_INF)
    dbp = jnp.pad(jnp.transpose(default_boxes, (1, 0)),
                  ((0, 0), (0, _PAD - _NUM_PRIORS)))

    num, npos = pl.pallas_call(
        _loss_kernel,
        grid=(_BATCH,),
        in_specs=[
            pl.BlockSpec((1, 4, _PAD), lambda b: (b, 0, 0)),
            pl.BlockSpec((1, _CPAD, _PAD), lambda b: (b, 0, 0)),
            pl.BlockSpec((1, _NUM_GT, 4), lambda b: (b, 0, 0)),
            pl.BlockSpec((4, _PAD), lambda b: (0, 0)),
        ],
        out_specs=[
            pl.BlockSpec((1, 1), lambda b: (0, 0)),
            pl.BlockSpec((1, 1), lambda b: (0, 0)),
        ],
        out_shape=[
            jax.ShapeDtypeStruct((1, 1), jnp.float32),
            jax.ShapeDtypeStruct((1, 1), jnp.float32),
        ],
        scratch_shapes=[
            pltpu.VMEM((_BATCH, _PAD), jnp.float32),
            pltpu.VMEM((_BATCH, 128), jnp.float32),
        ],
        interpret=interpret,
    )(locp, confp, gt_boxes, dbp)

    return num[0, 0] / (npos[0, 0] + 1e-6)


def kernel(loc_preds, conf_preds, gt_boxes, gt_labels, default_boxes):
    del gt_labels
    return _ssd_loss(loc_preds, conf_preds, gt_boxes, default_boxes)


# speedup vs baseline: 36.8184x; 1.1991x over previous
import functools

import jax
import jax.numpy as jnp
from jax.experimental import pallas as pl
from jax.experimental.pallas import tpu as pltpu

_NUM_CLASSES = 21
_NUM_PRIORS = 8732
_BATCH = 32
_NUM_GT = 10
_PAD = 8832
_CPAD = 24
_NEG_INF = -1e30


def _loss_kernel(loc_ref, conf_ref, gt_ref, db_ref, num_ref, npos_ref,
                 closs_ref, kcol_ref):
    b = pl.program_id(0)

    lane = jax.lax.broadcasted_iota(jnp.int32, (1, _PAD), 1)
    lane_valid = lane < _NUM_PRIORS

    db = db_ref[...]
    dbx0 = db[0:1, :]
    dby0 = db[1:2, :]
    dbx1 = db[2:3, :]
    dby1 = db[3:4, :]

    gt = gt_ref[0]
    gx0 = gt[:, 0:1]
    gy0 = gt[:, 1:2]
    gx1 = gt[:, 2:3]
    gy1 = gt[:, 3:4]

    area_d = (dbx1 - dbx0) * (dby1 - dby0)
    area_g = (gx1 - gx0) * (gy1 - gy0)
    ltx = jnp.maximum(dbx0, gx0)
    lty = jnp.maximum(dby0, gy0)
    rbx = jnp.minimum(dbx1, gx1)
    rby = jnp.minimum(dby1, gy1)
    iw = jnp.maximum(rbx - ltx, 0.0)
    ih = jnp.maximum(rby - lty, 0.0)
    inter = iw * ih
    union = area_d + area_g - inter
    iou = inter / union
    iou = jnp.where(lane_valid, iou, -1.0)

    g_col = jax.lax.broadcasted_iota(jnp.int32, (_NUM_GT, _PAD), 0)
    p_row = jax.lax.broadcasted_iota(jnp.int32, (_NUM_GT, _PAD), 1)

    best_ov = jnp.max(iou, axis=0, keepdims=True)
    bt_idx = jnp.min(jnp.where(iou == best_ov, g_col, _NUM_GT),
                     axis=0, keepdims=True)

    m_g = jnp.max(iou, axis=1, keepdims=True)
    bpi = jnp.min(jnp.where(iou == m_g, p_row, jnp.int32(1 << 30)),
                  axis=1, keepdims=True)

    ov_hit = p_row == bpi
    ocand = jnp.max(jnp.where(ov_hit, g_col, -1), axis=0, keepdims=True)
    bt_idx = jnp.where(ocand >= 0, ocand, bt_idx)
    best_ov = jnp.where(ocand >= 0, 2.0, best_ov)

    pos = best_ov >= 0.5
    npos_i = jnp.sum(pos.astype(jnp.int32))

    sel_f = (g_col == bt_idx).astype(jnp.float32)
    zf = jnp.zeros((), jnp.float32)
    gtT = jnp.transpose(gt, (1, 0))
    x0r = gtT[0:1, :]
    y0r = gtT[1:2, :]
    x1r = gtT[2:3, :]
    y1r = gtT[3:4, :]
    cw = jnp.concatenate([(x0r + x1r) * 0.5, (y0r + y1r) * 0.5,
                          x1r - x0r, y1r - y0r], axis=0)
    m_cw = jnp.dot(cw, sel_f, preferred_element_type=jnp.float32)

    d_w = dbx1 - dbx0
    d_h = dby1 - dby0
    d_cx = dbx0 + d_w * 0.5
    d_cy = dby0 + d_h * 0.5
    dwe = d_w + 1e-8
    dhe = d_h + 1e-8
    t0 = (m_cw[0:1, :] - d_cx) / dwe
    t1 = (m_cw[1:2, :] - d_cy) / dhe
    t2 = jnp.log(m_cw[2:3, :] / dwe)
    t3 = jnp.log(m_cw[3:4, :] / dhe)

    lp = loc_ref[0]

    def _sl1(d):
        return jnp.where(d < 1.0, 0.5 * d * d, d - 0.5)

    sl1 = (_sl1(jnp.abs(lp[0:1, :] - t0)) + _sl1(jnp.abs(lp[1:2, :] - t1))
           + _sl1(jnp.abs(lp[2:3, :] - t2)) + _sl1(jnp.abs(lp[3:4, :] - t3)))
    loc_loss = jnp.sum(jnp.where(pos, sl1, zf))

    cf = conf_ref[0]
    cmax = jnp.max(cf, axis=0, keepdims=True)
    ex = jnp.exp(cf - cmax)
    ssum = jnp.dot(jnp.ones((1, _CPAD), jnp.float32), ex,
                   preferred_element_type=jnp.float32)
    lse = cmax + jnp.log(ssum)
    closs = lse - cf[0:1, :]
    closs = jnp.where(jnp.logical_or(pos, jnp.logical_not(lane_valid)),
                      zf, closs)

    np_img = npos_i.astype(jnp.float32)
    closs_ref[pl.ds(b, 1), :] = closs
    kcol_ref[pl.ds(b, 1), :] = jnp.broadcast_to(jnp.reshape(np_img, (1, 1)),
                                                (1, 128))

    num2 = jnp.reshape(loc_loss, (1, 1))
    np2 = jnp.reshape(np_img, (1, 1))

    @pl.when(b == 0)
    def _init():
        num_ref[:, :] = num2
        npos_ref[:, :] = np2

    @pl.when(b != 0)
    def _acc():
        num_ref[:, :] += num2
        npos_ref[:, :] += np2

    @pl.when(b == _BATCH - 1)
    def _neg_mine():
        v = closs_ref[...]
        vi = jax.lax.bitcast_convert_type(v, jnp.int32)
        npv = kcol_ref[:, 0:1].astype(jnp.int32)
        k = jnp.minimum(3 * npv, _NUM_PRIORS - 1)

        def _body(_, lohi):
            lo, hi = lohi
            mid = lo + (hi - lo) // 2
            cnt = jnp.sum((vi > mid).astype(jnp.int32), axis=1, keepdims=True)
            big = cnt >= k
            return (jnp.where(big, mid, lo), jnp.where(big, hi, mid))

        lo0 = jnp.full((_BATCH, 1), -1, jnp.int32)
        hi0 = jnp.full((_BATCH, 1), 0x7F800000, jnp.int32)
        _, kth = jax.lax.fori_loop(0, 31, _body, (lo0, hi0))
        gt_mask = vi > kth
        cnt_gt = jnp.sum(gt_mask.astype(jnp.int32), axis=1, keepdims=True)
        sum_gt = jnp.sum(jnp.where(gt_mask, v, 0.0), axis=1, keepdims=True)
        kth_f = jax.lax.bitcast_convert_type(kth, jnp.float32)
        neg = sum_gt + kth_f * (k - cnt_gt).astype(jnp.float32)
        neg = jnp.where(k > 0, neg, 0.0)
        num_ref[:, :] += jnp.reshape(jnp.sum(neg), (1, 1))


@functools.partial(jax.jit, static_argnames=("interpret",))
def _ssd_loss(loc_preds, conf_preds, gt_boxes, default_boxes, interpret=False):
    locp = jnp.transpose(loc_preds, (0, 2, 1))
    locp = jnp.pad(locp, ((0, 0), (0, 0), (0, _PAD - _NUM_PRIORS)))
    confp = jnp.transpose(conf_preds, (0, 2, 1))
    confp = jnp.pad(confp,
                    ((0, 0), (0, _CPAD - _NUM_CLASSES), (0, _PAD - _NUM_PRIORS)),
                    constant_values=_NEG_INF)
    dbp = jnp.pad(jnp.transpose(default_boxes, (1, 0)),
                  ((0, 0), (0, _PAD - _NUM_PRIORS)))

    num, npos = pl.pallas_call(
        _loss_kernel,
        grid=(_BATCH,),
        in_specs=[
            pl.BlockSpec((1, 4, _PAD), lambda b: (b, 0, 0)),
            pl.BlockSpec((1, _CPAD, _PAD), lambda b: (b, 0, 0)),
            pl.BlockSpec((1, _NUM_GT, 4), lambda b: (b, 0, 0)),
            pl.BlockSpec((4, _PAD), lambda b: (0, 0)),
        ],
        out_specs=[
            pl.BlockSpec((1, 1), lambda b: (0, 0)),
            pl.BlockSpec((1, 1), lambda b: (0, 0)),
        ],
        out_shape=[
            jax.ShapeDtypeStruct((1, 1), jnp.float32),
            jax.ShapeDtypeStruct((1, 1), jnp.float32),
        ],
        scratch_shapes=[
            pltpu.VMEM((_BATCH, _PAD), jnp.float32),
            pltpu.VMEM((_BATCH, 128), jnp.float32),
        ],
        interpret=interpret,
    )(locp, confp, gt_boxes, dbp)

    return num[0, 0] / (npos[0, 0] + 1e-6)


def kernel(loc_preds, conf_preds, gt_boxes, gt_labels, default_boxes):
    del gt_labels
    return _ssd_loss(loc_preds, conf_preds, gt_boxes, default_boxes)


# speedup vs baseline: 42.7064x; 1.1599x over previous
import functools

import jax
import jax.numpy as jnp
from jax.experimental import pallas as pl
from jax.experimental.pallas import tpu as pltpu

_NUM_CLASSES = 21
_NUM_PRIORS = 8732
_BATCH = 32
_NUM_GT = 10
_PAD = 8732
_CPAD = 21


def _loss_kernel(loc_ref, conf_ref, gt_ref, db_ref, num_ref, npos_ref,
                 closs_ref, kcol_ref):
    b = pl.program_id(0)

    lane = jax.lax.broadcasted_iota(jnp.int32, (1, _PAD), 1)
    lane_valid = lane < _NUM_PRIORS

    db = db_ref[...]
    dbx0 = db[0:1, :]
    dby0 = db[1:2, :]
    dbx1 = db[2:3, :]
    dby1 = db[3:4, :]

    gt = gt_ref[0]
    gx0 = gt[:, 0:1]
    gy0 = gt[:, 1:2]
    gx1 = gt[:, 2:3]
    gy1 = gt[:, 3:4]

    area_d = (dbx1 - dbx0) * (dby1 - dby0)
    area_g = (gx1 - gx0) * (gy1 - gy0)
    ltx = jnp.maximum(dbx0, gx0)
    lty = jnp.maximum(dby0, gy0)
    rbx = jnp.minimum(dbx1, gx1)
    rby = jnp.minimum(dby1, gy1)
    iw = jnp.maximum(rbx - ltx, 0.0)
    ih = jnp.maximum(rby - lty, 0.0)
    inter = iw * ih
    union = area_d + area_g - inter
    iou = inter / union
    iou = jnp.where(lane_valid, iou, -1.0)

    g_col = jax.lax.broadcasted_iota(jnp.int32, (_NUM_GT, _PAD), 0)
    p_row = jax.lax.broadcasted_iota(jnp.int32, (_NUM_GT, _PAD), 1)

    best_ov = jnp.max(iou, axis=0, keepdims=True)
    bt_idx = jnp.min(jnp.where(iou == best_ov, g_col, _NUM_GT),
                     axis=0, keepdims=True)

    m_g = jnp.max(iou, axis=1, keepdims=True)
    bpi = jnp.min(jnp.where(iou == m_g, p_row, jnp.int32(1 << 30)),
                  axis=1, keepdims=True)

    ov_hit = p_row == bpi
    ocand = jnp.max(jnp.where(ov_hit, g_col, -1), axis=0, keepdims=True)
    bt_idx = jnp.where(ocand >= 0, ocand, bt_idx)
    best_ov = jnp.where(ocand >= 0, 2.0, best_ov)

    pos = best_ov >= 0.5
    npos_i = jnp.sum(pos.astype(jnp.int32))

    sel_f = (g_col == bt_idx).astype(jnp.float32)
    zf = jnp.zeros((), jnp.float32)
    gtT = jnp.transpose(gt, (1, 0))
    x0r = gtT[0:1, :]
    y0r = gtT[1:2, :]
    x1r = gtT[2:3, :]
    y1r = gtT[3:4, :]
    cw = jnp.concatenate([(x0r + x1r) * 0.5, (y0r + y1r) * 0.5,
                          x1r - x0r, y1r - y0r], axis=0)
    m_cw = jnp.dot(cw, sel_f, preferred_element_type=jnp.float32)

    d_w = dbx1 - dbx0
    d_h = dby1 - dby0
    d_cx = dbx0 + d_w * 0.5
    d_cy = dby0 + d_h * 0.5
    dwe = d_w + 1e-8
    dhe = d_h + 1e-8
    t0 = (m_cw[0:1, :] - d_cx) / dwe
    t1 = (m_cw[1:2, :] - d_cy) / dhe
    t2 = jnp.log(m_cw[2:3, :] / dwe)
    t3 = jnp.log(m_cw[3:4, :] / dhe)

    lp = loc_ref[0]

    def _sl1(d):
        return jnp.where(d < 1.0, 0.5 * d * d, d - 0.5)

    sl1 = (_sl1(jnp.abs(lp[0:1, :] - t0)) + _sl1(jnp.abs(lp[1:2, :] - t1))
           + _sl1(jnp.abs(lp[2:3, :] - t2)) + _sl1(jnp.abs(lp[3:4, :] - t3)))
    loc_loss = jnp.sum(jnp.where(pos, sl1, zf))

    cf = conf_ref[0]
    cmax = jnp.max(cf, axis=0, keepdims=True)
    ex = jnp.exp(cf - cmax)
    ssum = jnp.dot(jnp.ones((1, _CPAD), jnp.float32), ex,
                   preferred_element_type=jnp.float32)
    lse = cmax + jnp.log(ssum)
    closs = lse - cf[0:1, :]
    closs = jnp.where(jnp.logical_or(pos, jnp.logical_not(lane_valid)),
                      zf, closs)

    np_img = npos_i.astype(jnp.float32)
    closs_ref[pl.ds(b, 1), :] = closs
    kcol_ref[pl.ds(b, 1), :] = jnp.broadcast_to(jnp.reshape(np_img, (1, 1)),
                                                (1, 128))

    num2 = jnp.reshape(loc_loss, (1, 1))
    np2 = jnp.reshape(np_img, (1, 1))

    @pl.when(b == 0)
    def _init():
        num_ref[:, :] = num2
        npos_ref[:, :] = np2

    @pl.when(b != 0)
    def _acc():
        num_ref[:, :] += num2
        npos_ref[:, :] += np2

    @pl.when(b == _BATCH - 1)
    def _neg_mine():
        v = closs_ref[...]
        vi = jax.lax.bitcast_convert_type(v, jnp.int32)
        npv = kcol_ref[:, 0:1].astype(jnp.int32)
        k = jnp.minimum(3 * npv, _NUM_PRIORS - 1)

        def _body(_, lohi):
            lo, hi = lohi
            mid = lo + (hi - lo) // 2
            cnt = jnp.sum((vi > mid).astype(jnp.int32), axis=1, keepdims=True)
            big = cnt >= k
            return (jnp.where(big, mid, lo), jnp.where(big, hi, mid))

        lo0 = jnp.full((_BATCH, 1), -1, jnp.int32)
        hi0 = jnp.full((_BATCH, 1), 0x7F800000, jnp.int32)
        _, kth = jax.lax.fori_loop(0, 31, _body, (lo0, hi0))
        gt_mask = vi > kth
        cnt_gt = jnp.sum(gt_mask.astype(jnp.int32), axis=1, keepdims=True)
        sum_gt = jnp.sum(jnp.where(gt_mask, v, 0.0), axis=1, keepdims=True)
        kth_f = jax.lax.bitcast_convert_type(kth, jnp.float32)
        neg = sum_gt + kth_f * (k - cnt_gt).astype(jnp.float32)
        neg = jnp.where(k > 0, neg, 0.0)
        num_ref[:, :] += jnp.reshape(jnp.sum(neg), (1, 1))


@functools.partial(jax.jit, static_argnames=("interpret",))
def _ssd_loss(loc_preds, conf_preds, gt_boxes, default_boxes, interpret=False):
    locp = jnp.transpose(loc_preds, (0, 2, 1))
    confp = jnp.transpose(conf_preds, (0, 2, 1))
    dbp = jnp.transpose(default_boxes, (1, 0))

    num, npos = pl.pallas_call(
        _loss_kernel,
        grid=(_BATCH,),
        in_specs=[
            pl.BlockSpec((1, 4, _PAD), lambda b: (b, 0, 0)),
            pl.BlockSpec((1, _CPAD, _PAD), lambda b: (b, 0, 0)),
            pl.BlockSpec((1, _NUM_GT, 4), lambda b: (b, 0, 0)),
            pl.BlockSpec((4, _PAD), lambda b: (0, 0)),
        ],
        out_specs=[
            pl.BlockSpec((1, 1), lambda b: (0, 0)),
            pl.BlockSpec((1, 1), lambda b: (0, 0)),
        ],
        out_shape=[
            jax.ShapeDtypeStruct((1, 1), jnp.float32),
            jax.ShapeDtypeStruct((1, 1), jnp.float32),
        ],
        scratch_shapes=[
            pltpu.VMEM((_BATCH, _PAD), jnp.float32),
            pltpu.VMEM((_BATCH, 128), jnp.float32),
        ],
        interpret=interpret,
    )(locp, confp, gt_boxes, dbp)

    return num[0, 0] / (npos[0, 0] + 1e-6)


def kernel(loc_preds, conf_preds, gt_boxes, gt_labels, default_boxes):
    del gt_labels
    return _ssd_loss(loc_preds, conf_preds, gt_boxes, default_boxes)
